# baseline (speedup 1.0000x reference)
import functools

import jax
import jax.numpy as jnp
import numpy as np
from jax import lax
from jax.experimental import pallas as pl
from jax.experimental.pallas import tpu as pltpu
from jax.experimental.pallas import tpu_sc as plsc

N = 50000
NUM_FEATURES = 3
NUM_THETAS = 32
BUMP_STEPS = 32
R = 1.1
NUM_SEGMENTS = 128
DELTA = 2.0 * R / (BUMP_STEPS - 1)

L = 16
CHUNK = 3200
NPAD = 51200
NCHUNK = NPAD // CHUNK
ITERS = CHUNK // L
HROWS = BUMP_STEPS + 1
HSTRIDE = HROWS * NUM_SEGMENTS + 1
PARW = 128
OUTW = BUMP_STEPS * NUM_SEGMENTS


def _sc_body(xt_hbm, idx_hbm, par_hbm, out_hbm,
             xb0, xb1, ib0, ib1, pv, hist, outb,
             sx0, sx1, si0, si1, sp):
    c_id = lax.axis_index("c")
    s_id = lax.axis_index("s")
    tid = s_id * 2 + c_id

    pltpu.async_copy(par_hbm.at[pl.ds(tid * PARW, PARW)], pv, sp).wait()

    @plsc.parallel_loop(0, L * HSTRIDE, step=L, unroll=8)
    def _zero(k):
        hist[pl.ds(k, L)] = jnp.zeros((L,), jnp.float32)

    v0 = pv[pl.ds(0, L)]
    v1 = pv[pl.ds(L, L)]
    v2 = pv[pl.ds(2 * L, L)]
    a = pv[pl.ds(3 * L, L)]
    c0 = pv[pl.ds(4 * L, L)]
    c1 = pv[pl.ds(5 * L, L)]
    lane_off = lax.iota(jnp.int32, L) * HSTRIDE
    half_bumps = jnp.float32(0.5 * (BUMP_STEPS - 1) + 0.5)

    xbufs, ibufs, sxs, sis = (xb0, xb1), (ib0, ib1), (sx0, sx1), (si0, si1)

    def _start(c):
        b = c % 2
        cx = pltpu.async_copy(
            xt_hbm.at[:, pl.ds(c * CHUNK, CHUNK)], xbufs[b], sxs[b])
        ci = pltpu.async_copy(
            idx_hbm.at[pl.ds(c * CHUNK, CHUNK)], ibufs[b], sis[b])
        return cx, ci

    def _process(c, masked):
        b = c % 2
        xb, ib = xbufs[b], ibufs[b]

        @plsc.parallel_loop(0, CHUNK, step=L, unroll=8)
        def _iter(i):
            sl = pl.ds(i, L)
            x0 = xb[0, sl]
            hist[pl.ds(0, L)] = x0

    pending = _start(0)
    pending[0].wait()
    pending[1].wait()
    _process(0, masked=False)

    def _prefix(j, run):
        new = []
        for c8 in range(NUM_SEGMENTS // L):
            acc = run[c8]
            for l in range(L):
                acc = acc + hist[pl.ds(l * HSTRIDE + j * NUM_SEGMENTS
                                       + c8 * L, L)]
            outb[pl.ds(j * NUM_SEGMENTS + c8 * L, L)] = acc
            new.append(acc)
        return tuple(new)

    lax.fori_loop(0, BUMP_STEPS, _prefix,
                  tuple(jnp.zeros((L,), jnp.float32)
                        for _ in range(NUM_SEGMENTS // L)))

    pltpu.sync_copy(outb, out_hbm.at[pl.ds(tid * OUTW, OUTW)])


@jax.jit
def kernel(x, index, v, scale):
    scale_f = jnp.asarray(scale, jnp.float32)
    c0 = scale_f * jnp.float32(DELTA)
    ones = jnp.ones((L,), jnp.float32)
    zeros32 = jnp.zeros((PARW - 6 * L,), jnp.float32)
    par = jnp.concatenate([
        jnp.concatenate([
            v[0, t] * scale_f * ones,
            v[1, t] * scale_f * ones,
            v[2, t] * scale_f * ones,
            (1.0 / c0) * ones,
            c0 * ones,
            (-scale_f * jnp.float32(R)) * ones,
            zeros32,
        ])
        for t in range(NUM_THETAS)
    ])
    xt = jnp.broadcast_to(x[:1, :1], (3, NPAD)) + 0.0
    idxp = jnp.pad(index, (0, NPAD - N), constant_values=NUM_SEGMENTS)

    mesh = plsc.VectorSubcoreMesh(core_axis_name="c", subcore_axis_name="s")
    outT = pl.kernel(
        _sc_body,
        out_type=jax.ShapeDtypeStruct((NUM_THETAS * OUTW,), jnp.float32),
        mesh=mesh,
        compiler_params=pltpu.CompilerParams(needs_layout_passes=False),
        scratch_types=[
            pltpu.VMEM((NUM_FEATURES, CHUNK), jnp.float32),
            pltpu.VMEM((NUM_FEATURES, CHUNK), jnp.float32),
            pltpu.VMEM((CHUNK,), jnp.int32),
            pltpu.VMEM((CHUNK,), jnp.int32),
            pltpu.VMEM((PARW,), jnp.float32),
            pltpu.VMEM((L * HSTRIDE,), jnp.float32),
            pltpu.VMEM((OUTW,), jnp.float32),
            pltpu.SemaphoreType.DMA,
            pltpu.SemaphoreType.DMA,
            pltpu.SemaphoreType.DMA,
            pltpu.SemaphoreType.DMA,
            pltpu.SemaphoreType.DMA,
        ],
    )(xt, idxp, par)
    return jnp.transpose(
        outT.reshape(NUM_THETAS, BUMP_STEPS, NUM_SEGMENTS), (2, 1, 0))


# speedup vs baseline: 1.3486x; 1.3486x over previous
import functools

import jax
import jax.numpy as jnp
import numpy as np
from jax.experimental import pallas as pl
from jax.experimental.pallas import tpu as pltpu

N = 50000
NUM_FEATURES = 3
NUM_THETAS = 32
BUMP_STEPS = 32
R = 1.1
NUM_SEGMENTS = 128
BT = BUMP_STEPS * NUM_THETAS

C = 2048


def _body(x_ref, idx_ref, vt_ref, lin_ref, out_ref):
    i = pl.program_id(0)
    nh = jnp.dot(x_ref[...], vt_ref[...], preferred_element_type=jnp.float32)
    u = lin_ref[...] - nh
    ecc = (1.0 + jnp.tanh(u)).astype(jnp.bfloat16)
    ids = idx_ref[0, 0, :]
    rows = jax.lax.broadcasted_iota(jnp.int32, (NUM_SEGMENTS, C), 0)
    onehot = (rows == ids[None, :]).astype(jnp.bfloat16) * jnp.bfloat16(0.5)
    part = jnp.dot(onehot, ecc, preferred_element_type=jnp.float32)

    @pl.when(i == 0)
    def _init():
        out_ref[...] = part

    @pl.when(i > 0)
    def _acc():
        out_ref[...] += part


@jax.jit
def kernel(x, index, v, scale):
    n = x.shape[0]
    npad = ((n + C - 1) // C) * C
    g = npad // C
    half_scale = jnp.asarray(scale, jnp.float32) * 0.5
    lin = np.linspace(-R, R, BUMP_STEPS, dtype=np.float32)
    lin2 = jnp.asarray(np.repeat(lin, NUM_THETAS).reshape(1, BT)) * half_scale
    vt = jnp.tile(v * half_scale, (1, BUMP_STEPS)).astype(jnp.bfloat16)
    xp = jnp.pad(x, ((0, npad - n), (0, 0))).astype(jnp.bfloat16)
    idxp = jnp.pad(index, (0, npad - n), constant_values=NUM_SEGMENTS)
    idx3 = idxp.reshape(g, 1, C)

    out = pl.pallas_call(
        _body,
        grid=(g,),
        in_specs=[
            pl.BlockSpec((C, NUM_FEATURES), lambda i: (i, 0)),
            pl.BlockSpec((1, 1, C), lambda i: (i, 0, 0)),
            pl.BlockSpec((NUM_FEATURES, BT), lambda i: (0, 0)),
            pl.BlockSpec((1, BT), lambda i: (0, 0)),
        ],
        out_specs=pl.BlockSpec((NUM_SEGMENTS, BT), lambda i: (0, 0)),
        out_shape=jax.ShapeDtypeStruct((NUM_SEGMENTS, BT), jnp.float32),
        compiler_params=pltpu.CompilerParams(
            dimension_semantics=("arbitrary",),
        ),
    )(xp, idx3, vt, lin2)
    return out.reshape(NUM_SEGMENTS, BUMP_STEPS, NUM_THETAS)
